# Initial kernel scaffold; baseline (speedup 1.0000x reference)
import jax
import jax.numpy as jnp
from jax.experimental import pallas as pl


def kernel(user_embs, pos_embs, neg_embs, hard_negatives, loss_mask, item_table):
    raise NotImplementedError("write your pallas kernel here")


# speedup vs baseline: 28.3056x; 28.3056x over previous
import functools

import jax
import jax.numpy as jnp
from jax import lax
from jax.experimental import pallas as pl
from jax.experimental.pallas import tpu as pltpu
from jax.experimental.pallas import tpu_sc as plsc

B, L, D = 1024, 20, 128
M = B * L
K_HARD = 16
T_TOTAL = 512
NUM_HARD = 153
NUM_INBATCH = T_TOTAL - NUM_HARD
INV_TEMP = 20.0
IDX_PAD = 368
N_GROUPS = IDX_PAD // 16
TAIL_VALID = NUM_INBATCH - (N_GROUPS - 1) * 16

_SC_INFO = plsc.get_sparse_core_info()
NC = _SC_INFO.num_cores
NS = _SC_INFO.num_subcores
NW = NC * NS


def _norm_body(q_ref, k_ref, qn_ref, qb_ref, kb_ref, l0_ref):
    q = q_ref[...]
    k = k_ref[...]
    qn = q / jnp.maximum(jnp.sqrt(jnp.sum(q * q, axis=1, keepdims=True)), 1e-12)
    kn = k / jnp.maximum(jnp.sqrt(jnp.sum(k * k, axis=1, keepdims=True)), 1e-12)
    qn_ref[...] = qn
    qb_ref[...] = qn.astype(jnp.bfloat16)
    kb_ref[...] = kn.astype(jnp.bfloat16)
    l0_ref[...] = jnp.sum(qn * kn, axis=1) * INV_TEMP


def _normalize(q, k):
    bm = 2048
    return pl.pallas_call(
        _norm_body,
        grid=(M // bm,),
        in_specs=[
            pl.BlockSpec((bm, D), lambda i: (i, 0)),
            pl.BlockSpec((bm, D), lambda i: (i, 0)),
        ],
        out_specs=[
            pl.BlockSpec((bm, D), lambda i: (i, 0)),
            pl.BlockSpec((bm, D), lambda i: (i, 0)),
            pl.BlockSpec((bm, D), lambda i: (i, 0)),
            pl.BlockSpec((bm,), lambda i: (i,)),
        ],
        out_shape=[
            jax.ShapeDtypeStruct((M, D), jnp.float32),
            jax.ShapeDtypeStruct((M, D), jnp.bfloat16),
            jax.ShapeDtypeStruct((M, D), jnp.bfloat16),
            jax.ShapeDtypeStruct((M,), jnp.float32),
        ],
    )(q, k)


def _matmul_body(a_ref, b_ref, o_ref):
    o_ref[...] = lax.dot_general(
        a_ref[...], b_ref[...],
        (((1,), (1,)), ((), ())),
        preferred_element_type=jnp.float32,
    ) * INV_TEMP


def _similarity(qb, kb):
    bm, bn = 1024, 1024
    return pl.pallas_call(
        _matmul_body,
        grid=(M // bm, M // bn),
        in_specs=[
            pl.BlockSpec((bm, D), lambda i, j: (i, 0)),
            pl.BlockSpec((bn, D), lambda i, j: (j, 0)),
        ],
        out_specs=pl.BlockSpec((bm, bn), lambda i, j: (i, j)),
        out_shape=jax.ShapeDtypeStruct((M, M), jnp.float32),
    )(qb, kb)


_G_TOTAL = M * K_HARD
_G_PER_W = _G_TOTAL // NW
_G_CHUNK = 128
_G_STEPS = _G_PER_W // _G_CHUNK


def _hard_gather_body(table_hbm, ids_hbm, out_hbm, idx_v, rows_v, sem):
    wid = lax.axis_index("s") * NC + lax.axis_index("c")
    wbase = wid * _G_PER_W

    def step(c, _):
        base = wbase + c * _G_CHUNK
        pltpu.sync_copy(ids_hbm.at[pl.ds(base, _G_CHUNK)], idx_v)
        pltpu.async_copy(table_hbm.at[idx_v], rows_v, sem).wait()
        pltpu.sync_copy(rows_v, out_hbm.at[pl.ds(base, _G_CHUNK)])
        return _

    lax.fori_loop(0, _G_STEPS, step, None)


def _hard_gather(item_table, ids_flat):
    mesh = plsc.VectorSubcoreMesh(core_axis_name="c", subcore_axis_name="s")
    f = pl.kernel(
        _hard_gather_body,
        out_type=jax.ShapeDtypeStruct((_G_TOTAL, D), jnp.float32),
        mesh=mesh,
        scratch_types=[
            pltpu.VMEM((_G_CHUNK,), jnp.int32),
            pltpu.VMEM((_G_CHUNK, D), jnp.float32),
            pltpu.SemaphoreType.DMA,
        ],
    )
    return f(item_table, ids_flat)


_R_PER_W = M // NW
_R_GRP = 64
_R_NGRP = _R_PER_W // _R_GRP


def _inbatch_body(s_hbm, idx_hbm, out_hbm, srow_v, idx_v, acc_v, sem):
    wid = lax.axis_index("s") * NC + lax.axis_index("c")
    wbase = wid * _R_PER_W
    iota16 = lax.iota(jnp.int32, 16)
    tail_mask = iota16 < TAIL_VALID

    def group(g, _):
        grp_base = wbase + g * _R_GRP
        pltpu.sync_copy(
            idx_hbm.at[pl.ds(grp_base * IDX_PAD, _R_GRP * IDX_PAD)], idx_v)

        def row(i, _):
            m = grp_base + i
            pltpu.sync_copy(s_hbm.at[m], srow_v)
            acc = jnp.zeros((16,), jnp.float32)
            for g16 in range(N_GROUPS):
                iv = idx_v[pl.ds(i * IDX_PAD + g16 * 16, 16)]
                vals = plsc.load_gather(srow_v, [iv])
                e = jnp.exp(vals)
                if g16 == N_GROUPS - 1:
                    e = jnp.where(tail_mask, e, 0.0)
                acc = acc + e
            acc_v[pl.ds(i * 16, 16)] = acc
            return _

        lax.fori_loop(0, _R_GRP, row, None)
        pltpu.sync_copy(acc_v, out_hbm.at[pl.ds(grp_base * 16, _R_GRP * 16)])
        return _

    lax.fori_loop(0, _R_NGRP, group, None)


def _inbatch_sums(s, idx_pad_flat):
    mesh = plsc.VectorSubcoreMesh(core_axis_name="c", subcore_axis_name="s")
    f = pl.kernel(
        _inbatch_body,
        out_type=jax.ShapeDtypeStruct((M * 16,), jnp.float32),
        mesh=mesh,
        scratch_types=[
            pltpu.VMEM((M,), jnp.float32),
            pltpu.VMEM((_R_GRP * IDX_PAD,), jnp.int32),
            pltpu.VMEM((_R_GRP * 16,), jnp.float32),
            pltpu.SemaphoreType.DMA,
        ],
        compiler_params=pltpu.CompilerParams(needs_layout_passes=False),
    )
    return f(s, idx_pad_flat)


def _final_body(qn_ref, h_ref, sidx_ref, l0_ref, inp_ref, mask_ref,
                num_ref, den_ref):
    @pl.when(pl.program_id(0) == 0)
    def _init():
        num_ref[...] = jnp.zeros((1, 1), jnp.float32)
        den_ref[...] = jnp.zeros((1, 1), jnp.float32)

    qn = qn_ref[...]
    h = h_ref[...]
    hn = h / jnp.maximum(
        jnp.sqrt(jnp.sum(h * h, axis=2, keepdims=True)), 1e-12)
    l16 = jnp.sum(qn[:, None, :] * hn, axis=2) * INV_TEMP
    e16 = jnp.exp(l16)
    sidx = sidx_ref[...]
    hard_sum = jnp.zeros(e16.shape[:1], jnp.float32)
    for j in range(K_HARD):
        cnt = jnp.sum((sidx == j).astype(jnp.float32), axis=1)
        hard_sum = hard_sum + cnt * e16[:, j]
    in_sum = jnp.sum(inp_ref[...], axis=1)
    l0 = l0_ref[...]
    mask = (mask_ref[...] > 0.5).astype(jnp.float32)
    loss = jnp.log(jnp.exp(l0) + hard_sum + in_sum) - l0
    num_ref[...] += jnp.sum(loss * mask).reshape(1, 1)
    den_ref[...] += jnp.sum(mask).reshape(1, 1)


def _finalize(qn, hard_rows, sample_idx, l0, in_part, mask_flat):
    bf = 1024
    return pl.pallas_call(
        _final_body,
        grid=(M // bf,),
        in_specs=[
            pl.BlockSpec((bf, D), lambda i: (i, 0)),
            pl.BlockSpec((bf, K_HARD, D), lambda i: (i, 0, 0)),
            pl.BlockSpec((bf, NUM_HARD), lambda i: (i, 0)),
            pl.BlockSpec((bf,), lambda i: (i,)),
            pl.BlockSpec((bf, 16), lambda i: (i, 0)),
            pl.BlockSpec((bf,), lambda i: (i,)),
        ],
        out_specs=[
            pl.BlockSpec((1, 1), lambda i: (0, 0)),
            pl.BlockSpec((1, 1), lambda i: (0, 0)),
        ],
        out_shape=[
            jax.ShapeDtypeStruct((1, 1), jnp.float32),
            jax.ShapeDtypeStruct((1, 1), jnp.float32),
        ],
    )(qn, hard_rows, sample_idx, l0, in_part, mask_flat)


def kernel(user_embs, pos_embs, neg_embs, hard_negatives, loss_mask, item_table):
    del neg_embs
    q = user_embs.reshape(M, D)
    k = pos_embs.reshape(M, D)
    ids_flat = hard_negatives.reshape(M * K_HARD)

    skey = jax.random.key(42)
    k1, k2 = jax.random.split(skey)
    sample_idx = jax.random.randint(k1, (M, NUM_HARD), 0, K_HARD)
    rows = jnp.arange(M)
    r = jax.random.randint(k2, (M, NUM_INBATCH), 0, M - 1)
    inbatch_idx = r + (r >= rows[:, None]).astype(r.dtype)
    idx_pad = jnp.pad(inbatch_idx.astype(jnp.int32), ((0, 0), (0, IDX_PAD - NUM_INBATCH)))

    qn, qb, kb, l0 = _normalize(q, k)
    hard_rows = _hard_gather(item_table, ids_flat)
    s = _similarity(qb, kb)
    in_part = _inbatch_sums(s, idx_pad.reshape(M * IDX_PAD))
    num, den = _finalize(
        qn,
        hard_rows.reshape(M, K_HARD, D),
        sample_idx.astype(jnp.int32),
        l0,
        in_part.reshape(M, 16),
        loss_mask.reshape(M),
    )
    return num[0, 0] / den[0, 0]
